# baseline (speedup 1.0000x reference)
import functools

import jax
import jax.numpy as jnp
from jax import lax
from jax.experimental import pallas as pl
from jax.experimental.pallas import tpu as pltpu
from jax.experimental.pallas import tpu_sc as plsc

D = 64
DP = 128
NW = 32
CHUNK = 128
NG = CHUNK // 16


def _transpose_block(src, dst, rows_j, par_j):
    for d in range(D):
        for j in range(NG):
            vals = plsc.load_gather(src, [rows_j[j], par_j[j] + d])
            dst[d, pl.ds(16 * j, 16)] = vals


def _build_gather(seq: int, batch: int):
    nb = batch // CHUNK
    assert nb == NW
    npair = seq // 2
    mesh = plsc.VectorSubcoreMesh(core_axis_name="c", subcore_axis_name="s")

    @functools.partial(
        pl.kernel,
        mesh=mesh,
        compiler_params=pltpu.CompilerParams(needs_layout_passes=False),
        out_type=jax.ShapeDtypeStruct((seq, D, batch), jnp.float32),
        scratch_types=[
            pltpu.VMEM((seq, CHUNK), jnp.int32),
            pltpu.VMEM((seq, CHUNK), jnp.int32),
            pltpu.VMEM((CHUNK, DP), jnp.float32),
            pltpu.VMEM((CHUNK, DP), jnp.float32),
            pltpu.VMEM((D, CHUNK), jnp.float32),
            pltpu.SemaphoreType.DMA,
            pltpu.SemaphoreType.DMA,
        ],
    )
    def emb(table_hbm, idxhi_hbm, idxlo_hbm, out_hbm,
            idx_v, par_v, buf0, buf1, buft, g0, g1):
        wid = lax.axis_index("s") * 2 + lax.axis_index("c")
        col = wid * CHUNK
        iota16 = lax.iota(jnp.int32, 16)
        rows_j = [iota16 + 16 * j for j in range(NG)]
        pltpu.sync_copy(idxhi_hbm.at[wid], idx_v)
        pltpu.sync_copy(idxlo_hbm.at[wid], par_v)
        pltpu.async_copy(table_hbm.at[idx_v.at[0]], buf0, g0)

        def halfstep(s, buf):
            par_j = [par_v[s, pl.ds(16 * j, 16)] for j in range(NG)]
            _transpose_block(buf, buft, rows_j, par_j)
            pltpu.sync_copy(buft, out_hbm.at[s, :, pl.ds(col, CHUNK)])

        def body(p, carry):
            sa = 2 * p
            sb = sa + 1
            pltpu.make_async_copy(table_hbm.at[idx_v.at[sa]], buf0, g0).wait()
            pltpu.async_copy(table_hbm.at[idx_v.at[sb]], buf1, g1)
            halfstep(sa, buf0)
            pltpu.make_async_copy(table_hbm.at[idx_v.at[sb]], buf1, g1).wait()

            @pl.when(p + 1 < npair)
            def _():
                pltpu.async_copy(table_hbm.at[idx_v.at[sb + 1]], buf0, g0)

            halfstep(sb, buf1)
            return carry

        lax.fori_loop(0, npair, body, 0)

    return emb


def kernel(input_ids, attention_mask, table):
    b, s = input_ids.shape
    table2 = table.reshape(table.shape[0] // 2, DP)
    ids_w = input_ids.T.reshape(s, NW, CHUNK).transpose(1, 0, 2).astype(jnp.int32)
    ids_hi = ids_w >> 1
    ids_lo = (ids_w & 1) * D
    out3 = _build_gather(s, b)(table2, ids_hi, ids_lo)
    return out3.transpose(2, 0, 1), attention_mask


# speedup vs baseline: 1.2333x; 1.2333x over previous
import functools

import jax
import jax.numpy as jnp
from jax import lax
from jax.experimental import pallas as pl
from jax.experimental.pallas import tpu as pltpu
from jax.experimental.pallas import tpu_sc as plsc

D = 64
DP = 128
NW = 32
CHUNK = 128
NG = CHUNK // 16


def _transpose_block(src, dst, rows_j, par_j):
    lag = 8
    pending = []
    for d in range(D):
        for j in range(NG):
            vals = plsc.load_gather(src, [rows_j[j], par_j[j] + d])
            pending.append((d, j, vals))
            if len(pending) > lag:
                dd, jj, v = pending.pop(0)
                dst[dd, pl.ds(16 * jj, 16)] = v
    for dd, jj, v in pending:
        dst[dd, pl.ds(16 * jj, 16)] = v


def _build_gather(seq: int, batch: int):
    nb = batch // CHUNK
    assert nb == NW
    npair = seq // 2
    mesh = plsc.VectorSubcoreMesh(core_axis_name="c", subcore_axis_name="s")

    @functools.partial(
        pl.kernel,
        mesh=mesh,
        compiler_params=pltpu.CompilerParams(needs_layout_passes=False),
        out_type=jax.ShapeDtypeStruct((seq, D, batch), jnp.float32),
        scratch_types=[
            pltpu.VMEM((seq, CHUNK), jnp.int32),
            pltpu.VMEM((seq, CHUNK), jnp.int32),
            pltpu.VMEM((CHUNK, DP), jnp.float32),
            pltpu.VMEM((CHUNK, DP), jnp.float32),
            pltpu.VMEM((D, CHUNK), jnp.float32),
            pltpu.SemaphoreType.DMA,
            pltpu.SemaphoreType.DMA,
        ],
    )
    def emb(table_hbm, idxhi_hbm, idxlo_hbm, out_hbm,
            idx_v, par_v, buf0, buf1, buft, g0, g1):
        wid = lax.axis_index("s") * 2 + lax.axis_index("c")
        col = wid * CHUNK
        iota16 = lax.iota(jnp.int32, 16)
        rows_j = [iota16 + 16 * j for j in range(NG)]
        pltpu.sync_copy(idxhi_hbm.at[wid], idx_v)
        pltpu.sync_copy(idxlo_hbm.at[wid], par_v)
        pltpu.async_copy(table_hbm.at[idx_v.at[0]], buf0, g0)

        def halfstep(s, buf):
            par_j = [par_v[s, pl.ds(16 * j, 16)] for j in range(NG)]
            _transpose_block(buf, buft, rows_j, par_j)
            pltpu.sync_copy(buft, out_hbm.at[s, :, pl.ds(col, CHUNK)])

        def body(p, carry):
            sa = 2 * p
            sb = sa + 1
            pltpu.make_async_copy(table_hbm.at[idx_v.at[sa]], buf0, g0).wait()
            pltpu.async_copy(table_hbm.at[idx_v.at[sb]], buf1, g1)
            halfstep(sa, buf0)
            pltpu.make_async_copy(table_hbm.at[idx_v.at[sb]], buf1, g1).wait()

            @pl.when(p + 1 < npair)
            def _():
                pltpu.async_copy(table_hbm.at[idx_v.at[sb + 1]], buf0, g0)

            halfstep(sb, buf1)
            return carry

        lax.fori_loop(0, npair, body, 0)

    return emb


def kernel(input_ids, attention_mask, table):
    b, s = input_ids.shape
    table2 = table.reshape(table.shape[0] // 2, DP)
    ids_w = input_ids.T.reshape(s, NW, CHUNK).transpose(1, 0, 2).astype(jnp.int32)
    ids_hi = ids_w >> 1
    ids_lo = (ids_w & 1) * D
    out3 = _build_gather(s, b)(table2, ids_hi, ids_lo)
    return out3.transpose(2, 0, 1), attention_mask


# speedup vs baseline: 1.2747x; 1.0336x over previous
import functools

import jax
import jax.numpy as jnp
from jax import lax
from jax.experimental import pallas as pl
from jax.experimental.pallas import tpu as pltpu
from jax.experimental.pallas import tpu_sc as plsc

D = 64
DP = 128
NW = 32
CHUNK = 128
NG = CHUNK // 16
NBUF = 2
NOUT = 2
UNROLL = 2


def _transpose_block(src, dst, rows_j, par_j):
    lag = 8
    pending = []
    for d in range(D):
        for j in range(NG):
            vals = plsc.load_gather(src, [rows_j[j], par_j[j] + d])
            pending.append((d, j, vals))
            if len(pending) > lag:
                dd, jj, v = pending.pop(0)
                dst[dd, pl.ds(16 * jj, 16)] = v
    for dd, jj, v in pending:
        dst[dd, pl.ds(16 * jj, 16)] = v


def _build_gather(seq: int, batch: int):
    nb = batch // CHUNK
    assert nb == NW
    nloop = seq // UNROLL
    assert seq == nloop * UNROLL
    mesh = plsc.VectorSubcoreMesh(core_axis_name="c", subcore_axis_name="s")

    @functools.partial(
        pl.kernel,
        mesh=mesh,
        compiler_params=pltpu.CompilerParams(needs_layout_passes=False),
        out_type=jax.ShapeDtypeStruct((seq, D, batch), jnp.float32),
        scratch_types=[
            pltpu.VMEM((seq, CHUNK), jnp.int32),
            pltpu.VMEM((seq, CHUNK), jnp.int32),
            [pltpu.VMEM((CHUNK, DP), jnp.float32) for _ in range(NBUF)],
            [pltpu.VMEM((D, CHUNK), jnp.float32) for _ in range(NOUT)],
            [pltpu.SemaphoreType.DMA for _ in range(NBUF)],
            [pltpu.SemaphoreType.DMA for _ in range(NOUT)],
        ],
    )
    def emb(table_hbm, idxhi_hbm, idxlo_hbm, out_hbm,
            idx_v, par_v, bufs, bufts, gsems, psems):
        wid = lax.axis_index("s") * 2 + lax.axis_index("c")
        col = wid * CHUNK
        iota16 = lax.iota(jnp.int32, 16)
        rows_j = [iota16 + 16 * j for j in range(NG)]
        pltpu.sync_copy(idxhi_hbm.at[wid], idx_v)
        pltpu.sync_copy(idxlo_hbm.at[wid], par_v)

        def gather(s, b):
            pltpu.async_copy(table_hbm.at[idx_v.at[s]], bufs[b], gsems[b])

        def gather_wait(s, b):
            pltpu.make_async_copy(
                table_hbm.at[idx_v.at[s]], bufs[b], gsems[b]).wait()

        def put(s, o):
            pltpu.async_copy(bufts[o], out_hbm.at[s, :, pl.ds(col, CHUNK)],
                             psems[o])

        def put_wait(s, o):
            pltpu.make_async_copy(
                bufts[o], out_hbm.at[s, :, pl.ds(col, CHUNK)], psems[o]).wait()

        def chunk_step(s, k, p, refill):
            b = k % NBUF
            o = k % NOUT
            gather_wait(s, b)

            @pl.when(p > 0)
            def _():
                put_wait(s - NOUT, o)
            par_j = [par_v[s, pl.ds(16 * j, 16)] for j in range(NG)]
            _transpose_block(bufs[b], bufts[o], rows_j, par_j)
            if refill == "static":
                gather(s + NBUF, b)
            elif refill == "guarded":
                @pl.when(s + NBUF < seq)
                def _():
                    gather(s + NBUF, b)
            put(s, o)

        for b in range(NBUF):
            gather(b, b)
        def body(p, carry):
            s0 = p * UNROLL
            for k in range(UNROLL):
                chunk_step(s0 + k, k, p, "guarded")
            return carry

        lax.fori_loop(0, nloop, body, 0)
        for o in range(NOUT):
            put_wait(seq - NOUT + o, (seq - NOUT + o) % NOUT)

    return emb


def kernel(input_ids, attention_mask, table):
    b, s = input_ids.shape
    table2 = table.reshape(table.shape[0] // 2, DP)
    ids_w = input_ids.T.reshape(s, NW, CHUNK).transpose(1, 0, 2).astype(jnp.int32)
    ids_hi = ids_w >> 1
    ids_lo = (ids_w & 1) * D
    out3 = _build_gather(s, b)(table2, ids_hi, ids_lo)
    return out3.transpose(2, 0, 1), attention_mask
